# baseline (speedup 1.0000x reference)
import dataclasses
import functools

import jax
import jax.numpy as jnp
from jax import lax
from jax.experimental import pallas as pl
from jax.experimental.pallas import tpu as pltpu
from jax.experimental.pallas import tpu_sc as plsc

NC, NS, L = 2, 16, 16
NW = NC * NS
B = 16384
D = 64
BPW = B // NW
CHUNK = 256


def _compiler_params():
    cp = pltpu.CompilerParams()
    fields = pltpu.CompilerParams.__dataclass_fields__
    if "needs_layout_passes" in fields:
        cp = dataclasses.replace(cp, needs_layout_passes=False)
    return cp


def kernel(user, item, user_table, item_table):
    upk = user_table.reshape(user_table.shape[0] // 2, 2 * D)
    ipk = item_table.reshape(item_table.shape[0] // 2, 2 * D)
    mesh = plsc.VectorSubcoreMesh(core_axis_name="c", subcore_axis_name="s")

    @functools.partial(
        pl.kernel,
        mesh=mesh,
        compiler_params=_compiler_params(),
        out_type=jax.ShapeDtypeStruct((B,), jnp.float32),
        scratch_types=[
            pltpu.VMEM((BPW,), jnp.int32),
            pltpu.VMEM((BPW,), jnp.int32),
            pltpu.VMEM((CHUNK,), jnp.int32),
            pltpu.VMEM((CHUNK,), jnp.int32),
            pltpu.VMEM((CHUNK, 2 * D), jnp.float32),
            pltpu.VMEM((CHUNK, 2 * D), jnp.float32),
            pltpu.VMEM((BPW,), jnp.float32),
            pltpu.SemaphoreType.DMA,
        ],
    )
    def k(user_hbm, item_hbm, ut_hbm, it_hbm, out_hbm,
          uidx_v, iidx_v, ush_v, ish_v, ubuf_v, ibuf_v, out_v, sem):
        wid = lax.axis_index("s") * NC + lax.axis_index("c")
        base = wid * BPW
        pltpu.sync_copy(user_hbm.at[pl.ds(base, BPW)], uidx_v)
        pltpu.sync_copy(item_hbm.at[pl.ds(base, BPW)], iidx_v)
        lanes = lax.iota(jnp.int32, L)

        @pl.loop(0, BPW, step=CHUNK)
        def _(c0):
            @pl.loop(0, CHUNK, step=L)
            def _(r):
                ush_v[pl.ds(r, L)] = uidx_v[pl.ds(c0 + r, L)] >> 1
                ish_v[pl.ds(r, L)] = iidx_v[pl.ds(c0 + r, L)] >> 1

            cu = pltpu.async_copy(ut_hbm.at[ush_v], ubuf_v, sem)
            ci = pltpu.async_copy(it_hbm.at[ish_v], ibuf_v, sem)
            cu.wait()
            ci.wait()

            @pl.loop(0, CHUNK, step=L)
            def _(g):
                out_vec = jnp.zeros((L,), jnp.float32)
                for j in range(L):
                    bg = jnp.full((L,), c0 + g + j, jnp.int32)
                    mu = (plsc.load_gather(uidx_v, [bg]) & 1) == 1
                    mi = (plsc.load_gather(iidx_v, [bg]) & 1) == 1
                    acc = jnp.zeros((L,), jnp.float32)
                    for t in range(D // L):
                        uu = jnp.where(
                            mu,
                            ubuf_v[g + j, pl.ds(D + t * L, L)],
                            ubuf_v[g + j, pl.ds(t * L, L)],
                        )
                        ii = jnp.where(
                            mi,
                            ibuf_v[g + j, pl.ds(D + t * L, L)],
                            ibuf_v[g + j, pl.ds(t * L, L)],
                        )
                        acc = acc + uu * ii
                    out_vec = jnp.where(lanes == j, jnp.sum(acc), out_vec)
                out_v[pl.ds(c0 + g, L)] = out_vec

        pltpu.sync_copy(out_v, out_hbm.at[pl.ds(base, BPW)])

    return k(user, item, upk, ipk)
